# baseline (speedup 1.0000x reference)
import functools

import jax
import jax.numpy as jnp
import numpy as np
from jax import lax
from jax.experimental import pallas as pl
from jax.experimental.pallas import tpu as pltpu
from jax.experimental.pallas import tpu_sc as plsc

N = 4096
S = 32
M = 8
H = 32
E = 32768
D1 = S * M
D2 = S * H
NTILES = 32
EBLK1 = 8192
NB1 = E // EBLK1
CH1 = 64
CH2 = 16
LCAP1 = EBLK1 + 2 * CH1
SUBCAP = EBLK1 + 32

_mesh = plsc.VectorSubcoreMesh(core_axis_name="c", subcore_axis_name="s")

_IOTA16 = None


def _zero_acc(acc, nrows, ncols):
    z = jnp.zeros((16,), jnp.float32)

    def body(i, carry):
        for j in range(ncols // 16):
            acc[i, pl.ds(j * 16, 16)] = z
        return carry

    lax.fori_loop(0, nrows, body, 0)


def _scan_block(b, lo, hi, src_hbm, dst_hbm, srcbuf, dstbuf, msrc, mdloc,
                eblk):
    pltpu.sync_copy(src_hbm.at[pl.ds(b * eblk, eblk)], srcbuf)
    pltpu.sync_copy(dst_hbm.at[pl.ds(b * eblk, eblk)], dstbuf)

    lo_v = jnp.full((16,), lo, jnp.int32)
    hi_v = jnp.full((16,), hi, jnp.int32)
    one_v = jnp.full((16,), 1, jnp.int32)
    zero_v = jnp.full((16,), 0, jnp.int32)

    def vec_body(k, cnt):
        for j in range(4):
            kk = k * 4 + j
            dv = dstbuf[pl.ds(kk * 16, 16)]
            sv = srcbuf[pl.ds(kk * 16, 16)]
            m = (dv >= lo_v) & (dv < hi_v)
            mi = jnp.where(m, one_v, zero_v)
            csum = plsc.cumsum(mi)
            pos = csum + jnp.full((16,), cnt - 1, jnp.int32)
            plsc.store_scatter(msrc, [pos], sv, mask=m)
            plsc.store_scatter(mdloc, [pos], dv - lo_v, mask=m)
            cnt = cnt + jnp.sum(mi)
        return cnt

    return lax.fori_loop(0, eblk // 64, vec_body, jnp.int32(0))


def _filter_sublist(msrc, mdloc, cnt, plo, subsrc, subdlo):
    iota = lax.iota(jnp.int32, 16)
    plo_v = jnp.full((16,), plo, jnp.int32)
    phi_v = jnp.full((16,), plo + 64, jnp.int32)
    one_v = jnp.full((16,), 1, jnp.int32)
    zero_v = jnp.full((16,), 0, jnp.int32)
    cnt_v = jnp.full((16,), cnt, jnp.int32)
    nvec = (cnt + 15) // 16

    def fbody(k, c2):
        for j in range(2):
            kk = k * 2 + j
            sv = msrc[pl.ds(kk * 16, 16)]
            dv = mdloc[pl.ds(kk * 16, 16)]
            valid = (iota + jnp.full((16,), kk * 16, jnp.int32)) < cnt_v
            m = valid & (dv >= plo_v) & (dv < phi_v)
            mi = jnp.where(m, one_v, zero_v)
            csum = plsc.cumsum(mi)
            pos = csum + jnp.full((16,), c2 - 1, jnp.int32)
            plsc.store_scatter(subsrc, [pos], sv, mask=m)
            plsc.store_scatter(subdlo, [pos], dv - plo_v, mask=m)
            c2 = c2 + jnp.sum(mi)
        return c2

    return lax.fori_loop(0, (nvec + 1) // 2, fbody, jnp.int32(0))


def _pad_list(msrc, cnt, padval, npad16):
    pv = jnp.full((16,), padval, jnp.int32)
    for k in range(npad16):
        msrc[pl.ds(cnt + k * 16, 16)] = pv


def _flush(table, msrc, mdloc, cnt, rows0, rows1, acc, sem0, sem1, ch,
           ngroups):
    nch = (cnt + (ch - 1)) // ch

    def start(ci, rows, sem):
        pltpu.async_copy(table.at[msrc.at[pl.ds(ci * ch, ch)]], rows, sem)

    def wait(ci, rows, sem):
        pltpu.make_async_copy(table.at[msrc.at[pl.ds(ci * ch, ch)]], rows,
                              sem).wait()

    def accum(ci, rows):
        nrows = jnp.minimum(ch, cnt - ci * ch)

        def one_row(r):
            li = ci * ch + r
            dspl = plsc.load_gather(mdloc, [jnp.full((16,), li, jnp.int32)])
            dloc = jnp.max(dspl)
            for gb in range(0, ngroups, 8):
                vals = [rows[r, pl.ds((gb + j) * 16, 16)] for j in range(8)]
                for j in range(8):
                    plsc.addupdate(acc.at[dloc, pl.ds((gb + j) * 16, 16)],
                                   vals[j])

        def pair_rows(r2, c2):
            one_row(r2 * 2)
            one_row(r2 * 2 + 1)
            return c2

        lax.fori_loop(0, nrows // 2, pair_rows, 0)

        @pl.when(nrows % 2 == 1)
        def _():
            one_row(nrows - 1)

    @pl.when(nch > 0)
    def _():
        start(0, rows0, sem0)

    def pair_body(pi, carry):
        ci0 = 2 * pi
        ci1 = ci0 + 1

        @pl.when(ci1 < nch)
        def _():
            start(ci1, rows1, sem1)

        wait(ci0, rows0, sem0)
        accum(ci0, rows0)

        @pl.when(ci0 + 2 < nch)
        def _():
            start(ci0 + 2, rows0, sem0)

        @pl.when(ci1 < nch)
        def _():
            wait(ci1, rows1, sem1)
            accum(ci1, rows1)

        return carry

    lax.fori_loop(0, (nch + 1) // 2, pair_body, 0)


@functools.partial(
    pl.kernel,
    out_type=(
        jax.ShapeDtypeStruct((N, D1), jnp.float32),
        jax.ShapeDtypeStruct((NTILES, NB1, 2, SUBCAP), jnp.int32),
        jax.ShapeDtypeStruct((NTILES, NB1, 2, SUBCAP), jnp.int32),
        jax.ShapeDtypeStruct((NTILES, NB1 * 2 * 16), jnp.int32),
    ),
    mesh=_mesh,
    scratch_types=[
        pltpu.VMEM((EBLK1,), jnp.int32),
        pltpu.VMEM((EBLK1,), jnp.int32),
        pltpu.VMEM((LCAP1,), jnp.int32),
        pltpu.VMEM((LCAP1,), jnp.int32),
        pltpu.VMEM((SUBCAP,), jnp.int32),
        pltpu.VMEM((SUBCAP,), jnp.int32),
        pltpu.VMEM((NB1 * 2 * 16,), jnp.int32),
        pltpu.VMEM((CH1, D1), jnp.float32),
        pltpu.VMEM((CH1, D1), jnp.float32),
        pltpu.VMEM((128, D1), jnp.float32),
        pltpu.SemaphoreType.DMA,
        pltpu.SemaphoreType.DMA,
    ],
    compiler_params=pltpu.CompilerParams(needs_layout_passes=False),
)
def _seg1(table, src_hbm, dst_hbm, out, lsrc, ldlo, counts, srcbuf, dstbuf,
          msrc, mdloc, subsrc, subdlo, cntbuf, rows0, rows1, acc, sem0,
          sem1):
    w = lax.axis_index("c") * 16 + lax.axis_index("s")
    lo = w * 128
    hi = lo + 128

    _zero_acc(acc, 128, D1)

    def block_body(b, carry):
        cnt = _scan_block(b, lo, hi, src_hbm, dst_hbm, srcbuf, dstbuf,
                          msrc, mdloc, EBLK1)
        _pad_list(msrc, cnt, lo, CH1 // 16)
        _flush(table, msrc, mdloc, cnt, rows0, rows1, acc, sem0, sem1,
               CH1, D1 // 16)
        for p in range(2):
            scnt = _filter_sublist(msrc, mdloc, cnt, p * 64, subsrc, subdlo)
            _pad_list(subsrc, scnt, lo, CH2 // 16)
            pltpu.sync_copy(subsrc, lsrc.at[w, b, p])
            pltpu.sync_copy(subdlo, ldlo.at[w, b, p])
            cntbuf[pl.ds((b * 2 + p) * 16, 16)] = jnp.full((16,), scnt,
                                                           jnp.int32)
        return carry

    lax.fori_loop(0, NB1, block_body, 0)
    pltpu.sync_copy(acc, out.at[pl.ds(lo, 128)])
    pltpu.sync_copy(cntbuf, counts.at[w])


@functools.partial(
    pl.kernel,
    out_type=jax.ShapeDtypeStruct((N, D2), jnp.float32),
    mesh=_mesh,
    scratch_types=[
        pltpu.VMEM((SUBCAP,), jnp.int32),
        pltpu.VMEM((SUBCAP,), jnp.int32),
        pltpu.VMEM((NB1 * 2 * 16,), jnp.int32),
        pltpu.VMEM((CH2, D2), jnp.float32),
        pltpu.VMEM((CH2, D2), jnp.float32),
        pltpu.VMEM((64, D2), jnp.float32),
        pltpu.SemaphoreType.DMA,
        pltpu.SemaphoreType.DMA,
    ],
    compiler_params=pltpu.CompilerParams(needs_layout_passes=False),
)
def _seg2(table, lsrc, ldlo, counts, out, gsrc, gdlo, cntbuf, rows0, rows1,
          acc, sem0, sem1):
    w = lax.axis_index("c") * 16 + lax.axis_index("s")
    pltpu.sync_copy(counts.at[w], cntbuf)

    for p in range(2):
        lo = w * 128 + p * 64

        _zero_acc(acc, 64, D2)

        def block_body(b, carry):
            pltpu.sync_copy(lsrc.at[w, b, p], gsrc)
            pltpu.sync_copy(ldlo.at[w, b, p], gdlo)
            cnt = jnp.max(cntbuf[pl.ds((b * 2 + p) * 16, 16)])
            _flush(table, gsrc, gdlo, cnt, rows0, rows1, acc, sem0, sem1,
                   CH2, D2 // 16)
            return carry

        lax.fori_loop(0, NB1, block_body, 0)
        pltpu.sync_copy(acc, out.at[pl.ds(lo, 64)])


def _mlp1_body(w_ref, a_ref, w1a_ref, b1a_ref, w1b_ref, b1b_ref, eps_ref,
               out_ref):
    x = (1.0 + eps_ref[0, 0]) * w_ref[...] + a_ref[...]
    w1a = w1a_ref[...]
    b1a = b1a_ref[...]
    w1b = w1b_ref[...]
    b1b = b1b_ref[...]
    for s in range(S):
        xs = x[:, s * M:(s + 1) * M]
        h = jnp.dot(xs, w1a, preferred_element_type=jnp.float32)
        h = jnp.maximum(h + b1a, 0.0)
        h = jnp.dot(h, w1b, preferred_element_type=jnp.float32)
        out_ref[:, s * H:(s + 1) * H] = jnp.maximum(h + b1b, 0.0)


def _mlp2_body(x_ref, a_ref, w2a_ref, b2a_ref, w2b_ref, b2b_ref, eps_ref,
               out_ref):
    g = (1.0 + eps_ref[0, 0]) * x_ref[...] + a_ref[...]
    w2a = w2a_ref[...]
    b2a = b2a_ref[...]
    w2b = w2b_ref[...]
    b2b = b2b_ref[...]
    acc = jnp.zeros((g.shape[0], H), jnp.float32)
    for s in range(S):
        gs = g[:, s * H:(s + 1) * H]
        r = jnp.dot(gs, w2a, preferred_element_type=jnp.float32)
        acc = acc + jnp.maximum(r + b2a, 0.0)
    out_ref[...] = (jnp.dot(acc, w2b, preferred_element_type=jnp.float32)
                    + float(S) * b2b)


_NB = 2048
_GRID = N // _NB


def _const(shape):
    return pl.BlockSpec(shape, lambda i: tuple(0 for _ in shape))


_mlp1 = pl.pallas_call(
    _mlp1_body,
    grid=(_GRID,),
    in_specs=[
        pl.BlockSpec((_NB, D1), lambda i: (i, 0)),
        pl.BlockSpec((_NB, D1), lambda i: (i, 0)),
        _const((M, H)),
        _const((1, H)),
        _const((H, H)),
        _const((1, H)),
        _const((1, 1)),
    ],
    out_specs=pl.BlockSpec((_NB, D2), lambda i: (i, 0)),
    out_shape=jax.ShapeDtypeStruct((N, D2), jnp.float32),
    compiler_params=pltpu.CompilerParams(dimension_semantics=("parallel",)),
)

_mlp2 = pl.pallas_call(
    _mlp2_body,
    grid=(_GRID,),
    in_specs=[
        pl.BlockSpec((_NB, D2), lambda i: (i, 0)),
        pl.BlockSpec((_NB, D2), lambda i: (i, 0)),
        _const((H, H)),
        _const((1, H)),
        _const((H, H)),
        _const((1, H)),
        _const((1, 1)),
    ],
    out_specs=pl.BlockSpec((_NB, H), lambda i: (i, 0)),
    out_shape=jax.ShapeDtypeStruct((N, H), jnp.float32),
    compiler_params=pltpu.CompilerParams(dimension_semantics=("parallel",)),
)


def kernel(W_list, edge_index, W1a, b1a, W1b, b1b, eps1, W2a, b2a, W2b, b2b,
           eps2):
    src = edge_index[0].astype(jnp.int32)
    dst = edge_index[1].astype(jnp.int32)

    table1 = W_list.reshape(N, D1)
    agg1, lsrc, ldlo, counts = _seg1(table1, src, dst)
    x2 = _mlp1(table1, agg1,
               W1a, b1a.reshape(1, H), W1b, b1b.reshape(1, H),
               eps1.reshape(1, 1))
    agg2 = _seg2(x2, lsrc, ldlo, counts)
    return _mlp2(x2, agg2,
                 W2a, b2a.reshape(1, H), W2b, b2b.reshape(1, H),
                 eps2.reshape(1, 1))


# speedup vs baseline: 1.0062x; 1.0062x over previous
import functools

import jax
import jax.numpy as jnp
import numpy as np
from jax import lax
from jax.experimental import pallas as pl
from jax.experimental.pallas import tpu as pltpu
from jax.experimental.pallas import tpu_sc as plsc

N = 4096
S = 32
M = 8
H = 32
E = 32768
D1 = S * M
D2 = S * H
NTILES = 32
EBLK1 = 8192
NB1 = E // EBLK1
CH1 = 64
CH2 = 16
LCAP1 = EBLK1 + 2 * CH1
SUBCAP = EBLK1 + 32

_mesh = plsc.VectorSubcoreMesh(core_axis_name="c", subcore_axis_name="s")

_IOTA16 = None


def _zero_acc(acc, nrows, ncols):
    z = jnp.zeros((16,), jnp.float32)

    def body(i, carry):
        for j in range(ncols // 16):
            acc[i, pl.ds(j * 16, 16)] = z
        return carry

    lax.fori_loop(0, nrows, body, 0)


def _scan_block(b, lo, hi, src_hbm, dst_hbm, srcbuf, dstbuf, msrc, mdloc,
                eblk):
    pltpu.sync_copy(src_hbm.at[pl.ds(b * eblk, eblk)], srcbuf)
    pltpu.sync_copy(dst_hbm.at[pl.ds(b * eblk, eblk)], dstbuf)

    lo_v = jnp.full((16,), lo, jnp.int32)
    hi_v = jnp.full((16,), hi, jnp.int32)
    one_v = jnp.full((16,), 1, jnp.int32)
    zero_v = jnp.full((16,), 0, jnp.int32)

    def vec_body(k, cnt):
        for j in range(4):
            kk = k * 4 + j
            dv = dstbuf[pl.ds(kk * 16, 16)]
            sv = srcbuf[pl.ds(kk * 16, 16)]
            m = (dv >= lo_v) & (dv < hi_v)
            mi = jnp.where(m, one_v, zero_v)
            csum = plsc.cumsum(mi)
            pos = csum + jnp.full((16,), cnt - 1, jnp.int32)
            plsc.store_scatter(msrc, [pos], sv, mask=m)
            plsc.store_scatter(mdloc, [pos], dv - lo_v, mask=m)
            cnt = cnt + jnp.sum(mi)
        return cnt

    return lax.fori_loop(0, eblk // 64, vec_body, jnp.int32(0))


def _filter_sublist(msrc, mdloc, cnt, plo, subsrc, subdlo):
    iota = lax.iota(jnp.int32, 16)
    plo_v = jnp.full((16,), plo, jnp.int32)
    phi_v = jnp.full((16,), plo + 64, jnp.int32)
    one_v = jnp.full((16,), 1, jnp.int32)
    zero_v = jnp.full((16,), 0, jnp.int32)
    cnt_v = jnp.full((16,), cnt, jnp.int32)
    nvec = (cnt + 15) // 16

    def fbody(k, c2):
        for j in range(2):
            kk = k * 2 + j
            sv = msrc[pl.ds(kk * 16, 16)]
            dv = mdloc[pl.ds(kk * 16, 16)]
            valid = (iota + jnp.full((16,), kk * 16, jnp.int32)) < cnt_v
            m = valid & (dv >= plo_v) & (dv < phi_v)
            mi = jnp.where(m, one_v, zero_v)
            csum = plsc.cumsum(mi)
            pos = csum + jnp.full((16,), c2 - 1, jnp.int32)
            plsc.store_scatter(subsrc, [pos], sv, mask=m)
            plsc.store_scatter(subdlo, [pos], dv - plo_v, mask=m)
            c2 = c2 + jnp.sum(mi)
        return c2

    return lax.fori_loop(0, (nvec + 1) // 2, fbody, jnp.int32(0))


def _pad_list(msrc, cnt, padval, npad16):
    pv = jnp.full((16,), padval, jnp.int32)
    for k in range(npad16):
        msrc[pl.ds(cnt + k * 16, 16)] = pv


def _flush(table, msrc, mdloc, cnt, rows0, rows1, acc, sem0, sem1, ch,
           ngroups):
    nch = (cnt + (ch - 1)) // ch

    def start(ci, rows, sem):
        pltpu.async_copy(table.at[msrc.at[pl.ds(ci * ch, ch)]], rows, sem)

    def wait(ci, rows, sem):
        pltpu.make_async_copy(table.at[msrc.at[pl.ds(ci * ch, ch)]], rows,
                              sem).wait()

    def accum(ci, rows):
        nrows = jnp.minimum(ch, cnt - ci * ch)

        def row_body(r, c2):
            li = ci * ch + r
            dspl = plsc.load_gather(mdloc, [jnp.full((16,), li, jnp.int32)])
            dloc = jnp.max(dspl)
            for gb in range(0, ngroups, 8):
                vals = [rows[r, pl.ds((gb + j) * 16, 16)] for j in range(8)]
                for j in range(8):
                    plsc.addupdate(acc.at[dloc, pl.ds((gb + j) * 16, 16)],
                                   vals[j])
            return c2

        lax.fori_loop(0, nrows, row_body, 0)

    @pl.when(nch > 0)
    def _():
        start(0, rows0, sem0)

    def pair_body(pi, carry):
        ci0 = 2 * pi
        ci1 = ci0 + 1

        @pl.when(ci1 < nch)
        def _():
            start(ci1, rows1, sem1)

        wait(ci0, rows0, sem0)
        accum(ci0, rows0)

        @pl.when(ci0 + 2 < nch)
        def _():
            start(ci0 + 2, rows0, sem0)

        @pl.when(ci1 < nch)
        def _():
            wait(ci1, rows1, sem1)
            accum(ci1, rows1)

        return carry

    lax.fori_loop(0, (nch + 1) // 2, pair_body, 0)


@functools.partial(
    pl.kernel,
    out_type=(
        jax.ShapeDtypeStruct((N, D1), jnp.float32),
        jax.ShapeDtypeStruct((NTILES, NB1, 2, SUBCAP), jnp.int32),
        jax.ShapeDtypeStruct((NTILES, NB1, 2, SUBCAP), jnp.int32),
        jax.ShapeDtypeStruct((NTILES, NB1 * 2 * 16), jnp.int32),
    ),
    mesh=_mesh,
    scratch_types=[
        pltpu.VMEM((EBLK1,), jnp.int32),
        pltpu.VMEM((EBLK1,), jnp.int32),
        pltpu.VMEM((LCAP1,), jnp.int32),
        pltpu.VMEM((LCAP1,), jnp.int32),
        pltpu.VMEM((SUBCAP,), jnp.int32),
        pltpu.VMEM((SUBCAP,), jnp.int32),
        pltpu.VMEM((NB1 * 2 * 16,), jnp.int32),
        pltpu.VMEM((CH1, D1), jnp.float32),
        pltpu.VMEM((CH1, D1), jnp.float32),
        pltpu.VMEM((128, D1), jnp.float32),
        pltpu.SemaphoreType.DMA,
        pltpu.SemaphoreType.DMA,
    ],
    compiler_params=pltpu.CompilerParams(needs_layout_passes=False),
)
def _seg1(table, src_hbm, dst_hbm, out, lsrc, ldlo, counts, srcbuf, dstbuf,
          msrc, mdloc, subsrc, subdlo, cntbuf, rows0, rows1, acc, sem0,
          sem1):
    w = lax.axis_index("c") * 16 + lax.axis_index("s")
    lo = w * 128
    hi = lo + 128

    _zero_acc(acc, 128, D1)

    def block_body(b, carry):
        cnt = _scan_block(b, lo, hi, src_hbm, dst_hbm, srcbuf, dstbuf,
                          msrc, mdloc, EBLK1)
        _pad_list(msrc, cnt, lo, CH1 // 16)
        _flush(table, msrc, mdloc, cnt, rows0, rows1, acc, sem0, sem1,
               CH1, D1 // 16)
        for p in range(2):
            scnt = _filter_sublist(msrc, mdloc, cnt, p * 64, subsrc, subdlo)
            _pad_list(subsrc, scnt, lo, CH2 // 16)
            pltpu.sync_copy(subsrc, lsrc.at[w, b, p])
            pltpu.sync_copy(subdlo, ldlo.at[w, b, p])
            cntbuf[pl.ds((b * 2 + p) * 16, 16)] = jnp.full((16,), scnt,
                                                           jnp.int32)
        return carry

    lax.fori_loop(0, NB1, block_body, 0)
    pltpu.sync_copy(acc, out.at[pl.ds(lo, 128)])
    pltpu.sync_copy(cntbuf, counts.at[w])


@functools.partial(
    pl.kernel,
    out_type=jax.ShapeDtypeStruct((N, D2), jnp.float32),
    mesh=_mesh,
    scratch_types=[
        pltpu.VMEM((SUBCAP,), jnp.int32),
        pltpu.VMEM((SUBCAP,), jnp.int32),
        pltpu.VMEM((NB1 * 2 * 16,), jnp.int32),
        pltpu.VMEM((CH2, D2), jnp.float32),
        pltpu.VMEM((CH2, D2), jnp.float32),
        pltpu.VMEM((64, D2), jnp.float32),
        pltpu.SemaphoreType.DMA,
        pltpu.SemaphoreType.DMA,
    ],
    compiler_params=pltpu.CompilerParams(needs_layout_passes=False),
)
def _seg2(table, lsrc, ldlo, counts, out, gsrc, gdlo, cntbuf, rows0, rows1,
          acc, sem0, sem1):
    w = lax.axis_index("c") * 16 + lax.axis_index("s")
    pltpu.sync_copy(counts.at[w], cntbuf)

    for p in range(2):
        lo = w * 128 + p * 64

        _zero_acc(acc, 64, D2)

        def block_body(b, carry):
            pltpu.sync_copy(lsrc.at[w, b, p], gsrc)
            pltpu.sync_copy(ldlo.at[w, b, p], gdlo)
            cnt = jnp.max(cntbuf[pl.ds((b * 2 + p) * 16, 16)])
            _flush(table, gsrc, gdlo, cnt, rows0, rows1, acc, sem0, sem1,
                   CH2, D2 // 16)
            return carry

        lax.fori_loop(0, NB1, block_body, 0)
        pltpu.sync_copy(acc, out.at[pl.ds(lo, 64)])


def _mlp1_body(w_ref, a_ref, w1a_ref, b1a_ref, w1b_ref, b1b_ref, eps_ref,
               out_ref):
    x = (1.0 + eps_ref[0, 0]) * w_ref[...] + a_ref[...]
    w1a = w1a_ref[...]
    b1a = b1a_ref[...]
    w1b = w1b_ref[...]
    b1b = b1b_ref[...]
    for s in range(S):
        xs = x[:, s * M:(s + 1) * M]
        h = jnp.dot(xs, w1a, preferred_element_type=jnp.float32)
        h = jnp.maximum(h + b1a, 0.0)
        h = jnp.dot(h, w1b, preferred_element_type=jnp.float32)
        out_ref[:, s * H:(s + 1) * H] = jnp.maximum(h + b1b, 0.0)


def _mlp2_body(x_ref, a_ref, w2a_ref, b2a_ref, w2b_ref, b2b_ref, eps_ref,
               out_ref):
    g = (1.0 + eps_ref[0, 0]) * x_ref[...] + a_ref[...]
    w2a = w2a_ref[...]
    b2a = b2a_ref[...]
    w2b = w2b_ref[...]
    b2b = b2b_ref[...]
    acc = jnp.zeros((g.shape[0], H), jnp.float32)
    for s in range(S):
        gs = g[:, s * H:(s + 1) * H]
        r = jnp.dot(gs, w2a, preferred_element_type=jnp.float32)
        acc = acc + jnp.maximum(r + b2a, 0.0)
    out_ref[...] = (jnp.dot(acc, w2b, preferred_element_type=jnp.float32)
                    + float(S) * b2b)


_NB = 2048
_GRID = N // _NB


def _const(shape):
    return pl.BlockSpec(shape, lambda i: tuple(0 for _ in shape))


_mlp1 = pl.pallas_call(
    _mlp1_body,
    grid=(_GRID,),
    in_specs=[
        pl.BlockSpec((_NB, D1), lambda i: (i, 0)),
        pl.BlockSpec((_NB, D1), lambda i: (i, 0)),
        _const((M, H)),
        _const((1, H)),
        _const((H, H)),
        _const((1, H)),
        _const((1, 1)),
    ],
    out_specs=pl.BlockSpec((_NB, D2), lambda i: (i, 0)),
    out_shape=jax.ShapeDtypeStruct((N, D2), jnp.float32),
    compiler_params=pltpu.CompilerParams(dimension_semantics=("parallel",)),
)

_mlp2 = pl.pallas_call(
    _mlp2_body,
    grid=(_GRID,),
    in_specs=[
        pl.BlockSpec((_NB, D2), lambda i: (i, 0)),
        pl.BlockSpec((_NB, D2), lambda i: (i, 0)),
        _const((H, H)),
        _const((1, H)),
        _const((H, H)),
        _const((1, H)),
        _const((1, 1)),
    ],
    out_specs=pl.BlockSpec((_NB, H), lambda i: (i, 0)),
    out_shape=jax.ShapeDtypeStruct((N, H), jnp.float32),
    compiler_params=pltpu.CompilerParams(dimension_semantics=("parallel",)),
)


def kernel(W_list, edge_index, W1a, b1a, W1b, b1b, eps1, W2a, b2a, W2b, b2b,
           eps2):
    src = edge_index[0].astype(jnp.int32)
    dst = edge_index[1].astype(jnp.int32)

    table1 = W_list.reshape(N, D1)
    agg1, lsrc, ldlo, counts = _seg1(table1, src, dst)
    x2 = _mlp1(table1, agg1,
               W1a, b1a.reshape(1, H), W1b, b1b.reshape(1, H),
               eps1.reshape(1, 1))
    agg2 = _seg2(x2, lsrc, ldlo, counts)
    return _mlp2(x2, agg2,
                 W2a, b2a.reshape(1, H), W2b, b2b.reshape(1, H),
                 eps2.reshape(1, 1))
